# baseline (speedup 1.0000x reference)
import functools

import jax
import jax.numpy as jnp
from jax import lax
from jax.experimental import pallas as pl
from jax.experimental.pallas import tpu as pltpu
from jax.experimental.pallas import tpu_sc as plsc

N = 1
M = 2048
DIM = 1024
QK = 1024
H = 16
TOPK = 8
D = 64
SCALE = QK ** (-0.5)
NH = 3 * H

BQ = 512
NW = 32
ROWS = H * M
RPT = ROWS // NW
CH = 128


def _qkv_body(x_ref, w3_ref, b_ref, w_ref, qb_ref, out_ref, xc_ref):
    c = pl.program_id(0)

    @pl.when(c == 0)
    def _():
        x = x_ref[...]
        xb = x.astype(jnp.bfloat16).astype(jnp.float32)
        zero = jnp.zeros((1, DIM), jnp.float32)
        xm1 = jnp.concatenate([zero, xb[:-1, :]], axis=0)
        xp1 = jnp.concatenate([xb[1:, :], zero], axis=0)
        w3 = w3_ref[...]
        conv = xm1 * w3[0:1, :] + xb * w3[1:2, :] + xp1 * w3[2:3, :]
        xc_ref[...] = x + (conv + b_ref[...])

    out_ref[0] = jax.lax.dot_general(
        xc_ref[...], w_ref[0], (((1,), (1,)), ((), ())),
        preferred_element_type=jnp.float32) + qb_ref[0]


def _qkv(x2d, w3, pos_b, qkv_w3, qkv_b3):
    return pl.pallas_call(
        _qkv_body,
        grid=(NH,),
        in_specs=[
            pl.BlockSpec((M, DIM), lambda c: (0, 0)),
            pl.BlockSpec((3, DIM), lambda c: (0, 0)),
            pl.BlockSpec((1, DIM), lambda c: (0, 0)),
            pl.BlockSpec((1, D, DIM), lambda c: (c, 0, 0)),
            pl.BlockSpec((1, 1, D), lambda c: (c, 0, 0)),
        ],
        out_specs=pl.BlockSpec((1, M, D), lambda c: (c, 0, 0)),
        out_shape=jax.ShapeDtypeStruct((NH, M, D), jnp.float32),
        scratch_shapes=[pltpu.VMEM((M, DIM), jnp.float32)],
    )(x2d, w3, pos_b.reshape(1, DIM), qkv_w3, qkv_b3)


def _route_body(h0, q_ref, k_ref, w_ref, idx_ref):
    h = pl.program_id(0) + h0
    q = q_ref[0]
    k = k_ref[0]
    logits = jax.lax.dot_general(
        k, q, (((1,), (1,)), ((), ())),
        preferred_element_type=jnp.float32) * SCALE
    half = M // 2
    neg = jnp.float32(-jnp.inf)
    lhs = logits[:half, :]
    rhs = logits[half:, :]
    il = lax.broadcasted_iota(jnp.int32, (half, BQ), 0)
    gt = lhs >= rhs
    fold = jnp.maximum(lhs, rhs)
    lose = jnp.minimum(lhs, rhs)
    widx = jnp.where(gt, il, il + half)
    lidx = jnp.where(gt, il + half, il)
    vals = []
    idxs = []
    for _ in range(TOPK):
        mx = jnp.max(fold, axis=0, keepdims=True)
        eq = fold == mx
        ij = jnp.min(jnp.where(eq, widx, M), axis=0, keepdims=True)
        hit = widx == ij
        fold = jnp.where(hit, lose, fold)
        widx = jnp.where(hit, lidx, widx)
        lose = jnp.where(hit, neg, lose)
        vals.append(mx)
        idxs.append(ij)
    v8 = jnp.concatenate(vals, axis=0)
    i8 = jnp.concatenate(idxs, axis=0)
    e = jnp.exp(v8 - v8[0:1, :])
    w8 = e / jnp.sum(e, axis=0, keepdims=True)
    w_ref[0] = jnp.transpose(w8, (1, 0))
    idx_ref[0] = jnp.transpose(i8 + h * M, (1, 0))


def _route(qkv_heads, h0, nh):
    return pl.pallas_call(
        functools.partial(_route_body, h0),
        grid=(nh, M // BQ),
        in_specs=[
            pl.BlockSpec((1, BQ, D), lambda h, i: (h0 + h, i, 0)),
            pl.BlockSpec((1, M, D), lambda h, i: (H + h0 + h, 0, 0)),
        ],
        out_specs=[
            pl.BlockSpec((1, BQ, TOPK), lambda h, i: (h, i, 0)),
            pl.BlockSpec((1, BQ, TOPK), lambda h, i: (h, i, 0)),
        ],
        out_shape=[
            jax.ShapeDtypeStruct((nh, M, TOPK), jnp.float32),
            jax.ShapeDtypeStruct((nh, M, TOPK), jnp.int32),
        ],
    )(qkv_heads, qkv_heads)


def _sc_body(rpt, v_hbm, idx_hbm, w_hbm, out_hbm, idx_v, rows_v, w_v, out_v,
             sem):
    wid = lax.axis_index("s") * 2 + lax.axis_index("c")
    base = wid * rpt

    def chunk(ci, _):
        start = pl.multiple_of(base + ci * CH, CH)
        pltpu.sync_copy(idx_hbm.at[pl.ds(pl.multiple_of(start * TOPK // 128, 8),
                                         CH * TOPK // 128)],
                        idx_v)
        pltpu.sync_copy(w_hbm.at[pl.ds(pl.multiple_of(start * TOPK, 8),
                                       CH * TOPK)],
                        w_v.at[pl.ds(0, CH * TOPK)])
        ng = CH * TOPK // 128
        for g in range(ng):
            pltpu.async_copy(v_hbm.at[idx_v.at[g]],
                             rows_v.at[pl.ds(g * 128, 128)], sem)
        for g in range(ng):
            pltpu.make_async_copy(v_hbm.at[idx_v.at[g]],
                                  rows_v.at[pl.ds(g * 128, 128)], sem).wait()

        def row(r, _):
            wrow = w_v[pl.ds(r * TOPK, 16)]
            for t in range(D // 16):
                acc = jnp.zeros((16,), jnp.float32)
                for j in range(TOPK):
                    acc = acc + wrow[j] * rows_v[r * TOPK + j,
                                                 pl.ds(t * 16, 16)]
                out_v[r, pl.ds(t * 16, 16)] = acc
            return 0

        lax.fori_loop(0, CH, row, 0)
        pltpu.sync_copy(out_v, out_hbm.at[pl.ds(start, CH)])
        return 0

    lax.fori_loop(0, rpt // CH, chunk, 0)


def _sc_gather(v_flat, idx3, w3):
    rows = idx3.shape[0] * idx3.shape[1]
    mesh = plsc.VectorSubcoreMesh(core_axis_name="c", subcore_axis_name="s",
                                  num_cores=2, num_subcores=16)
    f = pl.kernel(
        functools.partial(_sc_body, rows // NW),
        out_type=jax.ShapeDtypeStruct((rows, D), jnp.float32),
        mesh=mesh,
        scratch_types=[
            pltpu.VMEM((CH * TOPK // 128, 128), jnp.int32),
            pltpu.VMEM((CH * TOPK, D), jnp.float32),
            pltpu.VMEM((CH * TOPK + 8,), jnp.float32),
            pltpu.VMEM((CH, D), jnp.float32),
            pltpu.SemaphoreType.DMA,
        ],
        compiler_params=pltpu.CompilerParams(use_tc_tiling_on_sc=False),
    )
    return f(v_flat, idx3.reshape(rows * TOPK // 128, 128),
             w3.reshape(rows * TOPK))


def _wo_body(a_ref, w_ref, b_ref, out_ref):
    h = pl.program_id(0)

    @pl.when(h == 0)
    def _():
        out_ref[...] = jnp.broadcast_to(b_ref[...], (M, DIM))

    out_ref[...] += jax.lax.dot_general(
        a_ref[0], w_ref[0], (((1,), (0,)), ((), ())),
        preferred_element_type=jnp.float32)


def _wo(heads, wo_wt3, wo_b):
    return pl.pallas_call(
        _wo_body,
        grid=(H,),
        in_specs=[
            pl.BlockSpec((1, M, D), lambda h: (h, 0, 0)),
            pl.BlockSpec((1, D, DIM), lambda h: (h, 0, 0)),
            pl.BlockSpec((1, DIM), lambda h: (0, 0)),
        ],
        out_specs=pl.BlockSpec((M, DIM), lambda h: (0, 0)),
        out_shape=jax.ShapeDtypeStruct((M, DIM), jnp.float32),
    )(heads, wo_wt3, wo_b.reshape(1, DIM))


def kernel(x, pos_w, pos_b, qkv_w, qkv_b, wo_w, wo_b):
    x2d = x.reshape(M, DIM)
    w3 = jnp.transpose(pos_w[:, 0, :], (1, 0))
    qkv_heads = _qkv(x2d, w3, pos_b, qkv_w.reshape(NH, D, DIM),
                     qkv_b.reshape(NH, 1, D))
    v_flat = qkv_heads[2 * H:].reshape(ROWS, D)
    parts = []
    nh = H // 8
    for h0 in range(0, H, nh):
        w8, i8 = _route(qkv_heads, h0, nh)
        parts.append(_sc_gather(v_flat, i8, w8))
    out_flat = jnp.concatenate(parts, axis=0)
    out = _wo(out_flat.reshape(H, M, D), wo_w.T.reshape(H, D, DIM), wo_b)
    return out.reshape(N, M, DIM)


# speedup vs baseline: 1.0197x; 1.0197x over previous
import functools

import jax
import jax.numpy as jnp
from jax import lax
from jax.experimental import pallas as pl
from jax.experimental.pallas import tpu as pltpu
from jax.experimental.pallas import tpu_sc as plsc

N = 1
M = 2048
DIM = 1024
QK = 1024
H = 16
TOPK = 8
D = 64
SCALE = QK ** (-0.5)
NH = 3 * H

BQ = 512
NW = 32
ROWS = H * M
RPT = ROWS // NW
CH = 128


def _qkv_body(x_ref, w3_ref, b_ref, w_ref, qb_ref, out_ref, xc_ref):
    c = pl.program_id(0)

    @pl.when(c == 0)
    def _():
        x = x_ref[...]
        xb = x.astype(jnp.bfloat16).astype(jnp.float32)
        zero = jnp.zeros((1, DIM), jnp.float32)
        xm1 = jnp.concatenate([zero, xb[:-1, :]], axis=0)
        xp1 = jnp.concatenate([xb[1:, :], zero], axis=0)
        w3 = w3_ref[...]
        conv = xm1 * w3[0:1, :] + xb * w3[1:2, :] + xp1 * w3[2:3, :]
        xc_ref[...] = x + (conv + b_ref[...])

    out_ref[0] = jax.lax.dot_general(
        xc_ref[...], w_ref[0], (((1,), (1,)), ((), ())),
        preferred_element_type=jnp.float32) + qb_ref[0]


def _qkv(x2d, w3, pos_b, qkv_w3, qkv_b3):
    return pl.pallas_call(
        _qkv_body,
        grid=(NH,),
        in_specs=[
            pl.BlockSpec((M, DIM), lambda c: (0, 0)),
            pl.BlockSpec((3, DIM), lambda c: (0, 0)),
            pl.BlockSpec((1, DIM), lambda c: (0, 0)),
            pl.BlockSpec((1, D, DIM), lambda c: (c, 0, 0)),
            pl.BlockSpec((1, 1, D), lambda c: (c, 0, 0)),
        ],
        out_specs=pl.BlockSpec((1, M, D), lambda c: (c, 0, 0)),
        out_shape=jax.ShapeDtypeStruct((NH, M, D), jnp.float32),
        scratch_shapes=[pltpu.VMEM((M, DIM), jnp.float32)],
    )(x2d, w3, pos_b.reshape(1, DIM), qkv_w3, qkv_b3)


def _route_body(h0, q_ref, k_ref, w_ref, idx_ref):
    h = pl.program_id(0) + h0
    q = q_ref[0]
    k = k_ref[0]
    logits = jax.lax.dot_general(
        k, q, (((1,), (1,)), ((), ())),
        preferred_element_type=jnp.float32) * SCALE
    half = M // 2
    neg = jnp.float32(-jnp.inf)
    lhs = logits[:half, :]
    rhs = logits[half:, :]
    il = lax.broadcasted_iota(jnp.int32, (half, BQ), 0)
    gt = lhs >= rhs
    fold = jnp.maximum(lhs, rhs)
    lose = jnp.minimum(lhs, rhs)
    widx = jnp.where(gt, il, il + half)
    vals = []
    idxs = []
    for _ in range(TOPK):
        mx = jnp.max(fold, axis=0, keepdims=True)
        eq = fold == mx
        ij = jnp.min(jnp.where(eq, widx, M), axis=0, keepdims=True)
        hit = widx == ij
        fold = jnp.where(hit, lose, fold)
        widx = jnp.where(hit, widx ^ half, widx)
        lose = jnp.where(hit, neg, lose)
        vals.append(mx)
        idxs.append(ij)
    v8 = jnp.concatenate(vals, axis=0)
    i8 = jnp.concatenate(idxs, axis=0)
    e = jnp.exp(v8 - v8[0:1, :])
    w8 = e / jnp.sum(e, axis=0, keepdims=True)
    w_ref[0] = jnp.transpose(w8, (1, 0))
    idx_ref[0] = jnp.transpose(i8 + h * M, (1, 0))


def _route(qkv_heads, h0, nh):
    return pl.pallas_call(
        functools.partial(_route_body, h0),
        grid=(nh, M // BQ),
        in_specs=[
            pl.BlockSpec((1, BQ, D), lambda h, i: (h0 + h, i, 0)),
            pl.BlockSpec((1, M, D), lambda h, i: (H + h0 + h, 0, 0)),
        ],
        out_specs=[
            pl.BlockSpec((1, BQ, TOPK), lambda h, i: (h, i, 0)),
            pl.BlockSpec((1, BQ, TOPK), lambda h, i: (h, i, 0)),
        ],
        out_shape=[
            jax.ShapeDtypeStruct((nh, M, TOPK), jnp.float32),
            jax.ShapeDtypeStruct((nh, M, TOPK), jnp.int32),
        ],
    )(qkv_heads, qkv_heads)


def _sc_body(rpt, v_hbm, idx_hbm, w_hbm, out_hbm, idx_v, rows_v, w_v, out_v,
             sem):
    wid = lax.axis_index("s") * 2 + lax.axis_index("c")
    base = wid * rpt

    def chunk(ci, _):
        start = pl.multiple_of(base + ci * CH, CH)
        pltpu.sync_copy(idx_hbm.at[pl.ds(pl.multiple_of(start * TOPK // 128, 8),
                                         CH * TOPK // 128)],
                        idx_v)
        pltpu.sync_copy(w_hbm.at[pl.ds(pl.multiple_of(start * TOPK, 8),
                                       CH * TOPK)],
                        w_v.at[pl.ds(0, CH * TOPK)])
        ng = CH * TOPK // 128
        for g in range(ng):
            pltpu.async_copy(v_hbm.at[idx_v.at[g]],
                             rows_v.at[pl.ds(g * 128, 128)], sem)
        for g in range(ng):
            pltpu.make_async_copy(v_hbm.at[idx_v.at[g]],
                                  rows_v.at[pl.ds(g * 128, 128)], sem).wait()

        def row(r, _):
            wrow = w_v[pl.ds(r * TOPK, 16)]
            for t in range(D // 16):
                acc = jnp.zeros((16,), jnp.float32)
                for j in range(TOPK):
                    acc = acc + wrow[j] * rows_v[r * TOPK + j,
                                                 pl.ds(t * 16, 16)]
                out_v[r, pl.ds(t * 16, 16)] = acc
            return 0

        lax.fori_loop(0, CH, row, 0)
        pltpu.sync_copy(out_v, out_hbm.at[pl.ds(start, CH)])
        return 0

    lax.fori_loop(0, rpt // CH, chunk, 0)


def _sc_gather(v_flat, idx3, w3):
    rows = idx3.shape[0] * idx3.shape[1]
    mesh = plsc.VectorSubcoreMesh(core_axis_name="c", subcore_axis_name="s",
                                  num_cores=2, num_subcores=16)
    f = pl.kernel(
        functools.partial(_sc_body, rows // NW),
        out_type=jax.ShapeDtypeStruct((rows, D), jnp.float32),
        mesh=mesh,
        scratch_types=[
            pltpu.VMEM((CH * TOPK // 128, 128), jnp.int32),
            pltpu.VMEM((CH * TOPK, D), jnp.float32),
            pltpu.VMEM((CH * TOPK + 8,), jnp.float32),
            pltpu.VMEM((CH, D), jnp.float32),
            pltpu.SemaphoreType.DMA,
        ],
        compiler_params=pltpu.CompilerParams(use_tc_tiling_on_sc=False),
    )
    return f(v_flat, idx3.reshape(rows * TOPK // 128, 128),
             w3.reshape(rows * TOPK))


def _wo_body(a_ref, w_ref, b_ref, out_ref):
    h = pl.program_id(0)

    @pl.when(h == 0)
    def _():
        out_ref[...] = jnp.broadcast_to(b_ref[...], (M, DIM))

    out_ref[...] += jax.lax.dot_general(
        a_ref[0], w_ref[0], (((1,), (0,)), ((), ())),
        preferred_element_type=jnp.float32)


def _wo(heads, wo_wt3, wo_b):
    return pl.pallas_call(
        _wo_body,
        grid=(H,),
        in_specs=[
            pl.BlockSpec((1, M, D), lambda h: (h, 0, 0)),
            pl.BlockSpec((1, D, DIM), lambda h: (h, 0, 0)),
            pl.BlockSpec((1, DIM), lambda h: (0, 0)),
        ],
        out_specs=pl.BlockSpec((M, DIM), lambda h: (0, 0)),
        out_shape=jax.ShapeDtypeStruct((M, DIM), jnp.float32),
    )(heads, wo_wt3, wo_b.reshape(1, DIM))


def kernel(x, pos_w, pos_b, qkv_w, qkv_b, wo_w, wo_b):
    x2d = x.reshape(M, DIM)
    w3 = jnp.transpose(pos_w[:, 0, :], (1, 0))
    qkv_heads = _qkv(x2d, w3, pos_b, qkv_w.reshape(NH, D, DIM),
                     qkv_b.reshape(NH, 1, D))
    v_flat = qkv_heads[2 * H:].reshape(ROWS, D)
    parts = []
    nh = H // 4
    for h0 in range(0, H, nh):
        w8, i8 = _route(qkv_heads, h0, nh)
        parts.append(_sc_gather(v_flat, i8, w8))
    out_flat = jnp.concatenate(parts, axis=0)
    out = _wo(out_flat.reshape(H, M, D), wo_w.T.reshape(H, D, DIM), wo_b)
    return out.reshape(N, M, DIM)


# speedup vs baseline: 1.0341x; 1.0140x over previous
import functools

import jax
import jax.numpy as jnp
from jax import lax
from jax.experimental import pallas as pl
from jax.experimental.pallas import tpu as pltpu
from jax.experimental.pallas import tpu_sc as plsc

N = 1
M = 2048
DIM = 1024
QK = 1024
H = 16
TOPK = 8
D = 64
SCALE = QK ** (-0.5)
NH = 3 * H

BQ = 512
NW = 32
ROWS = H * M
RPT = ROWS // NW
CH = 64


def _qkv_body(x_ref, w3_ref, b_ref, w_ref, qb_ref, out_ref, xc_ref):
    c = pl.program_id(0)

    @pl.when(c == 0)
    def _():
        x = x_ref[...]
        xb = x.astype(jnp.bfloat16).astype(jnp.float32)
        zero = jnp.zeros((1, DIM), jnp.float32)
        xm1 = jnp.concatenate([zero, xb[:-1, :]], axis=0)
        xp1 = jnp.concatenate([xb[1:, :], zero], axis=0)
        w3 = w3_ref[...]
        conv = xm1 * w3[0:1, :] + xb * w3[1:2, :] + xp1 * w3[2:3, :]
        xc_ref[...] = x + (conv + b_ref[...])

    out_ref[0] = jax.lax.dot_general(
        xc_ref[...], w_ref[0], (((1,), (1,)), ((), ())),
        preferred_element_type=jnp.float32) + qb_ref[0]


def _qkv(x2d, w3, pos_b, qkv_w3, qkv_b3):
    return pl.pallas_call(
        _qkv_body,
        grid=(NH,),
        in_specs=[
            pl.BlockSpec((M, DIM), lambda c: (0, 0)),
            pl.BlockSpec((3, DIM), lambda c: (0, 0)),
            pl.BlockSpec((1, DIM), lambda c: (0, 0)),
            pl.BlockSpec((1, D, DIM), lambda c: (c, 0, 0)),
            pl.BlockSpec((1, 1, D), lambda c: (c, 0, 0)),
        ],
        out_specs=pl.BlockSpec((1, M, D), lambda c: (c, 0, 0)),
        out_shape=jax.ShapeDtypeStruct((NH, M, D), jnp.float32),
        scratch_shapes=[pltpu.VMEM((M, DIM), jnp.float32)],
    )(x2d, w3, pos_b.reshape(1, DIM), qkv_w3, qkv_b3)


def _route_body(h0, q_ref, k_ref, w_ref, idx_ref):
    h = pl.program_id(0) + h0
    q = q_ref[0]
    k = k_ref[0]
    logits = jax.lax.dot_general(
        k, q, (((1,), (1,)), ((), ())),
        preferred_element_type=jnp.float32) * SCALE
    half = M // 2
    neg = jnp.float32(-jnp.inf)
    lhs = logits[:half, :]
    rhs = logits[half:, :]
    il = lax.broadcasted_iota(jnp.int32, (half, BQ), 0)
    gt = lhs >= rhs
    fold = jnp.maximum(lhs, rhs)
    lose = jnp.minimum(lhs, rhs)
    widx = jnp.where(gt, il, il + half)
    lidx = jnp.where(gt, il + half, il)
    vals = []
    idxs = []
    for _ in range(TOPK):
        mx = jnp.max(fold, axis=0, keepdims=True)
        eq = fold == mx
        ij = jnp.min(jnp.where(eq, widx, M), axis=0, keepdims=True)
        hit = widx == ij
        fold = jnp.where(hit, lose, fold)
        widx = jnp.where(hit, lidx, widx)
        lose = jnp.where(hit, neg, lose)
        vals.append(mx)
        idxs.append(ij)
    v8 = jnp.concatenate(vals, axis=0)
    i8 = jnp.concatenate(idxs, axis=0)
    e = jnp.exp(v8 - v8[0:1, :])
    w8 = e / jnp.sum(e, axis=0, keepdims=True)
    w_ref[0] = jnp.transpose(w8, (1, 0))
    idx_ref[0] = jnp.transpose(i8 + h * M, (1, 0))


def _route(qkv_heads, h0, nh):
    return pl.pallas_call(
        functools.partial(_route_body, h0),
        grid=(nh, M // BQ),
        in_specs=[
            pl.BlockSpec((1, BQ, D), lambda h, i: (h0 + h, i, 0)),
            pl.BlockSpec((1, M, D), lambda h, i: (H + h0 + h, 0, 0)),
        ],
        out_specs=[
            pl.BlockSpec((1, BQ, TOPK), lambda h, i: (h, i, 0)),
            pl.BlockSpec((1, BQ, TOPK), lambda h, i: (h, i, 0)),
        ],
        out_shape=[
            jax.ShapeDtypeStruct((nh, M, TOPK), jnp.float32),
            jax.ShapeDtypeStruct((nh, M, TOPK), jnp.int32),
        ],
    )(qkv_heads, qkv_heads)


def _sc_body(rpt, v_hbm, idx_hbm, w_hbm, out_hbm, idx_v, rows_v, w_v, out_v,
             sems):
    wid = lax.axis_index("s") * 2 + lax.axis_index("c")
    base = wid * rpt
    nck = rpt // CH
    ng = CH * TOPK // 128

    def stage(ci, b):
        start = pl.multiple_of(base + ci * CH, CH)
        pltpu.sync_copy(idx_hbm.at[pl.ds(pl.multiple_of(start * TOPK // 128, 4),
                                         ng)],
                        idx_v.at[b])
        pltpu.sync_copy(w_hbm.at[pl.ds(pl.multiple_of(start * TOPK, 8),
                                       CH * TOPK)],
                        w_v.at[b, pl.ds(0, CH * TOPK)])
        for g in range(ng):
            pltpu.async_copy(v_hbm.at[idx_v.at[b, g]],
                             rows_v.at[b, pl.ds(g * 128, 128)], sems[b])

    stage(0, 0)
    for ci in range(nck):
        b = ci % 2
        start = pl.multiple_of(base + ci * CH, CH)
        for g in range(ng):
            pltpu.make_async_copy(v_hbm.at[idx_v.at[b, g]],
                                  rows_v.at[b, pl.ds(g * 128, 128)],
                                  sems[b]).wait()
        if ci + 1 < nck:
            stage(ci + 1, 1 - b)

        def row(r, _):
            wrow = w_v[b, pl.ds(r * TOPK, 16)]
            for t in range(D // 16):
                acc = jnp.zeros((16,), jnp.float32)
                for j in range(TOPK):
                    acc = acc + wrow[j] * rows_v[b, r * TOPK + j,
                                                 pl.ds(t * 16, 16)]
                out_v[r, pl.ds(t * 16, 16)] = acc
            return 0

        lax.fori_loop(0, CH, row, 0)
        pltpu.sync_copy(out_v, out_hbm.at[pl.ds(start, CH)])


def _sc_gather(v_flat, idx3, w3):
    rows = idx3.shape[0] * idx3.shape[1]
    mesh = plsc.VectorSubcoreMesh(core_axis_name="c", subcore_axis_name="s",
                                  num_cores=2, num_subcores=16)
    f = pl.kernel(
        functools.partial(_sc_body, rows // NW),
        out_type=jax.ShapeDtypeStruct((rows, D), jnp.float32),
        mesh=mesh,
        scratch_types=[
            pltpu.VMEM((2, CH * TOPK // 128, 128), jnp.int32),
            pltpu.VMEM((2, CH * TOPK, D), jnp.float32),
            pltpu.VMEM((2, CH * TOPK + 16), jnp.float32),
            pltpu.VMEM((CH, D), jnp.float32),
            [pltpu.SemaphoreType.DMA, pltpu.SemaphoreType.DMA],
        ],
        compiler_params=pltpu.CompilerParams(use_tc_tiling_on_sc=False),
    )
    return f(v_flat, idx3.reshape(rows * TOPK // 128, 128),
             w3.reshape(rows * TOPK))


def _wo_body(a_ref, w_ref, b_ref, out_ref):
    h = pl.program_id(0)

    @pl.when(h == 0)
    def _():
        out_ref[...] = jnp.broadcast_to(b_ref[...], (M, DIM))

    out_ref[...] += jax.lax.dot_general(
        a_ref[0], w_ref[0], (((1,), (0,)), ((), ())),
        preferred_element_type=jnp.float32)


def _wo(heads, wo_wt3, wo_b):
    return pl.pallas_call(
        _wo_body,
        grid=(H,),
        in_specs=[
            pl.BlockSpec((1, M, D), lambda h: (h, 0, 0)),
            pl.BlockSpec((1, D, DIM), lambda h: (h, 0, 0)),
            pl.BlockSpec((1, DIM), lambda h: (0, 0)),
        ],
        out_specs=pl.BlockSpec((M, DIM), lambda h: (0, 0)),
        out_shape=jax.ShapeDtypeStruct((M, DIM), jnp.float32),
    )(heads, wo_wt3, wo_b.reshape(1, DIM))


def kernel(x, pos_w, pos_b, qkv_w, qkv_b, wo_w, wo_b):
    x2d = x.reshape(M, DIM)
    w3 = jnp.transpose(pos_w[:, 0, :], (1, 0))
    qkv_heads = _qkv(x2d, w3, pos_b, qkv_w.reshape(NH, D, DIM),
                     qkv_b.reshape(NH, 1, D))
    v_flat = qkv_heads[2 * H:].reshape(ROWS, D)
    parts = []
    nh = H // 4
    for h0 in range(0, H, nh):
        w8, i8 = _route(qkv_heads, h0, nh)
        parts.append(_sc_gather(v_flat, i8, w8))
    out_flat = jnp.concatenate(parts, axis=0)
    out = _wo(out_flat.reshape(H, M, D), wo_w.T.reshape(H, D, DIM), wo_b)
    return out.reshape(N, M, DIM)
